# baseline (speedup 1.0000x reference)
import numpy as np
import jax
import jax.numpy as jnp
from jax.experimental import pallas as pl
from jax.experimental.pallas import tpu as pltpu

_N = 1000.0
_LOG_P1 = float(np.log(np.float32(1e-12)))
_LOG_N = float(np.log(np.float64(1000.0)))
_LOG_NP1 = float(np.log(np.float64(1001.0)))


def _lae(a, b):
    m = jnp.maximum(a, b)
    return m + jnp.log1p(jnp.exp(-jnp.abs(a - b)))


def _body(x_ref, q_ref, t_ref, o_ref):
    x = x_ref[...]
    q = q_ref[...]
    tf = t_ref[...].astype(jnp.float32)
    B, C = x.shape

    lane = jax.lax.broadcasted_iota(jnp.int32, (B, C), 1)
    even = (lane % 2) == 0

    def partner(v):
        return jnp.where(even, pltpu.roll(v, C - 1, 1), pltpu.roll(v, 1, 1))

    xp = partner(x)
    qp = partner(q)

    zs = _lae(x, xp)
    ls_own = x - zs
    ls_part = xp - zs
    zq = _lae(q, qp)
    lq_own = q - zq
    lq_part = qp - zq

    nt = _N - tf
    log_nt1 = jnp.log(nt + 1.0)
    la = jnp.log(nt) - log_nt1
    b = -log_nt1
    lca = jnp.where(tf >= 1.0, log_nt1, _LOG_N) - _LOG_NP1
    l1mca = jnp.log(jnp.maximum(tf, 1.0)) - _LOG_NP1
    bp = b + _LOG_P1

    a1 = jnp.where(even, bp + lq_part, la + lq_own)
    a2 = jnp.where(even, lq_own, b + lq_part)
    b1 = jnp.where(even, l1mca, lca + ls_own)
    b2 = jnp.where(even, lca + ls_own, l1mca + _LOG_P1)

    u = _lae(a1, a2) + _lae(b1, b2)
    up = partner(u)
    lse = _lae(u, up)
    o_ref[...] = u - lse


def kernel(log_x_start, log_x_t, t_edge):
    E = log_x_start.shape[0]
    C = 128
    R = (2 * E) // C
    X = log_x_start.reshape(R, C)
    Q = log_x_t.reshape(R, C)
    T = jnp.broadcast_to(t_edge.astype(jnp.int32)[:, None], (E, 2)).reshape(R, C)

    grid = 5 if R % 5 == 0 else 1
    br = R // grid
    spec = pl.BlockSpec((br, C), lambda i: (i, 0))
    o = pl.pallas_call(
        _body,
        grid=(grid,),
        in_specs=[spec] * 3,
        out_specs=spec,
        out_shape=jax.ShapeDtypeStruct((R, C), jnp.float32),
    )(X, Q, T)
    return o.reshape(E, 2)


# speedup vs baseline: 24.2842x; 24.2842x over previous
import numpy as np
import jax
import jax.numpy as jnp
from jax.experimental import pallas as pl

_N = 1000.0
_LOG_P1 = float(np.log(np.float32(1e-12)))
_LOG_N = float(np.log(np.float64(1000.0)))
_LOG_NP1 = float(np.log(np.float64(1001.0)))


def _lae(a, b):
    m = jnp.maximum(a, b)
    return m + jnp.log1p(jnp.exp(-jnp.abs(a - b)))


def _body(s0_ref, s1_ref, q0_ref, q1_ref, t_ref, o0_ref, o1_ref):
    s0 = s0_ref[...]
    s1 = s1_ref[...]
    q0 = q0_ref[...]
    q1 = q1_ref[...]
    tf = t_ref[...].astype(jnp.float32)

    zs = _lae(s0, s1)
    lxs0 = s0 - zs
    lxs1 = s1 - zs
    zq = _lae(q0, q1)
    lxt0 = q0 - zq
    lxt1 = q1 - zq

    nt = _N - tf
    log_nt1 = jnp.log(nt + 1.0)
    la = jnp.log(nt) - log_nt1
    b = -log_nt1
    lca = jnp.where(tf >= 1.0, log_nt1, _LOG_N) - _LOG_NP1
    l1mca = jnp.log(jnp.maximum(tf, 1.0)) - _LOG_NP1
    bp = b + _LOG_P1

    lx0_xt = _lae(bp + lxt1, lxt0)
    lx1_xt = _lae(la + lxt1, b + lxt0)
    lx0_xs = _lae(l1mca, lca + lxs0)
    lx1_xs = _lae(lca + lxs1, l1mca + _LOG_P1)

    u0 = lx0_xt + lx0_xs
    u1 = lx1_xt + lx1_xs
    lse = _lae(u0, u1)
    o0_ref[...] = u0 - lse
    o1_ref[...] = u1 - lse


def kernel(log_x_start, log_x_t, t_edge):
    E = log_x_start.shape[0]
    C = 128
    R = E // C
    packed = jnp.concatenate(
        [log_x_start[:, 0], log_x_start[:, 1], log_x_t[:, 0], log_x_t[:, 1]]
    ).reshape(4 * R, C)
    tt = t_edge.astype(jnp.int32).reshape(R, C)

    grid = 5 if R % 5 == 0 else 1
    br = R // grid
    nb = R // br
    spec = pl.BlockSpec((br, C), lambda i: (i, 0))

    def off(k):
        return pl.BlockSpec((br, C), lambda i, _k=k: (i + _k * nb, 0))

    o0, o1 = pl.pallas_call(
        _body,
        grid=(grid,),
        in_specs=[off(0), off(1), off(2), off(3), spec],
        out_specs=[spec, spec],
        out_shape=[jax.ShapeDtypeStruct((R, C), jnp.float32)] * 2,
    )(packed, packed, packed, packed, tt)
    return jnp.stack([o0.reshape(E), o1.reshape(E)], axis=1)


# speedup vs baseline: 26.8859x; 1.1071x over previous
import numpy as np
import jax
import jax.numpy as jnp
from jax.experimental import pallas as pl

_N = 1000.0
_LOG_P1 = float(np.log(np.float32(1e-12)))
_LOG_N = float(np.log(np.float64(1000.0)))
_LOG_NP1 = float(np.log(np.float64(1001.0)))


def _lae(a, b):
    m = jnp.maximum(a, b)
    return m + jnp.log1p(jnp.exp(-jnp.abs(a - b)))


def _nsp(d):
    return -(jnp.maximum(d, 0.0) + jnp.log1p(jnp.exp(-jnp.abs(d))))


def _body(s0_ref, s1_ref, q0_ref, q1_ref, t_ref, o0_ref, o1_ref):
    s0 = s0_ref[...]
    s1 = s1_ref[...]
    q0 = q0_ref[...]
    q1 = q1_ref[...]
    tf = t_ref[...].astype(jnp.float32)

    ds = s1 - s0
    lxs0 = _nsp(ds)
    lxs1 = lxs0 + ds
    dq = q1 - q0
    lxt0 = _nsp(dq)
    lxt1 = lxt0 + dq

    nt = _N - tf
    log_nt1 = jnp.log(nt + 1.0)
    la = jnp.log(nt) - log_nt1
    b = -log_nt1
    lca = jnp.where(tf >= 1.0, log_nt1, _LOG_N) - _LOG_NP1
    l1mca = jnp.log(jnp.maximum(tf, 1.0)) - _LOG_NP1
    bp = b + _LOG_P1

    lx0_xt = _lae(bp + lxt1, lxt0)
    lx1_xt = _lae(la + lxt1, b + lxt0)
    lx0_xs = _lae(l1mca, lca + lxs0)
    lx1_xs = _lae(lca + lxs1, l1mca + _LOG_P1)

    u0 = lx0_xt + lx0_xs
    u1 = lx1_xt + lx1_xs
    d = u1 - u0
    o0 = _nsp(d)
    o0_ref[...] = o0
    o1_ref[...] = o0 + d


def kernel(log_x_start, log_x_t, t_edge):
    E = log_x_start.shape[0]
    C = 128
    R = E // C
    s0 = log_x_start[:, 0].reshape(R, C)
    s1 = log_x_start[:, 1].reshape(R, C)
    q0 = log_x_t[:, 0].reshape(R, C)
    q1 = log_x_t[:, 1].reshape(R, C)
    tt = t_edge.astype(jnp.int32).reshape(R, C)

    grid = 25 if R % 25 == 0 else 1
    br = R // grid
    spec = pl.BlockSpec((br, C), lambda i: (i, 0))
    o0, o1 = pl.pallas_call(
        _body,
        grid=(grid,),
        in_specs=[spec] * 5,
        out_specs=[spec] * 2,
        out_shape=[jax.ShapeDtypeStruct((R, C), jnp.float32)] * 2,
    )(s0, s1, q0, q1, tt)
    return jnp.stack([o0.reshape(E), o1.reshape(E)], axis=1)


# speedup vs baseline: 32.1684x; 1.1965x over previous
import numpy as np
import jax
import jax.numpy as jnp
from jax.experimental import pallas as pl

_N = 1000.0
_LOG_P1 = float(np.log(np.float32(1e-12)))
_LOG_N = float(np.log(np.float64(1000.0)))
_LOG_NP1 = float(np.log(np.float64(1001.0)))


def _lae(a, b):
    m = jnp.maximum(a, b)
    return m + jnp.log1p(jnp.exp(-jnp.abs(a - b)))


def _nsp(d):
    return -(jnp.maximum(d, 0.0) + jnp.log1p(jnp.exp(-jnp.abs(d))))


def _body(s0_ref, s1_ref, q0_ref, q1_ref, t_ref, o0_ref, o1_ref):
    s0 = s0_ref[...]
    s1 = s1_ref[...]
    q0 = q0_ref[...]
    q1 = q1_ref[...]
    tf = t_ref[...].astype(jnp.float32)

    ds = s1 - s0
    lxs0 = _nsp(ds)
    lxs1 = lxs0 + ds
    dq = q1 - q0
    lxt0 = _nsp(dq)
    lxt1 = lxt0 + dq

    nt = _N - tf
    log_nt1 = jnp.log(nt + 1.0)
    la = jnp.log(nt) - log_nt1
    b = -log_nt1
    lca = jnp.where(tf >= 1.0, log_nt1, _LOG_N) - _LOG_NP1
    l1mca = jnp.log(jnp.maximum(tf, 1.0)) - _LOG_NP1
    bp = b + _LOG_P1

    lx0_xt = _lae(bp + lxt1, lxt0)
    lx1_xt = _lae(la + lxt1, b + lxt0)
    lx0_xs = _lae(l1mca, lca + lxs0)
    lx1_xs = _lae(lca + lxs1, l1mca + _LOG_P1)

    u0 = lx0_xt + lx0_xs
    u1 = lx1_xt + lx1_xs
    d = u1 - u0
    o0 = _nsp(d)
    o0_ref[...] = o0
    o1_ref[...] = o0 + d


def kernel(log_x_start, log_x_t, t_edge):
    E = log_x_start.shape[0]
    C = 128
    R = E // C
    s0 = log_x_start[:, 0].reshape(R, C)
    s1 = log_x_start[:, 1].reshape(R, C)
    q0 = log_x_t[:, 0].reshape(R, C)
    q1 = log_x_t[:, 1].reshape(R, C)
    tt = t_edge.astype(jnp.int32).reshape(R, C)

    grid = 5 if R % 5 == 0 else 1
    br = R // grid
    spec = pl.BlockSpec((br, C), lambda i: (i, 0))
    o0, o1 = pl.pallas_call(
        _body,
        grid=(grid,),
        in_specs=[spec] * 5,
        out_specs=[spec] * 2,
        out_shape=[jax.ShapeDtypeStruct((R, C), jnp.float32)] * 2,
    )(s0, s1, q0, q1, tt)
    return jnp.stack([o0.reshape(E), o1.reshape(E)], axis=1)
